# baseline (speedup 1.0000x reference)
import jax
import jax.numpy as jnp
from jax import lax
from jax.experimental import pallas as pl
from jax.experimental.pallas import tpu as pltpu

N_ROWS = 800000
N_AE = 32
N_DE = 16
N_SEG_ATOM = 50000

B = 1280
NB = N_ROWS // B
W = 128
TR = 50432
TE = 128
FS = 40

_C00 = (((0,), (0,)), ((), ()))
_BF = jnp.bfloat16
_F32 = jnp.float32


def _split_hi_lo(v):
    hi = v.astype(_BF)
    lo = (v - hi.astype(_F32)).astype(_BF)
    return hi, lo


def _k1a_body(lo_ref, hi_ref, x_ref, aidx_ref, eidx_ref, aacc_ref, eacc_ref):
    i = pl.program_id(0)

    @pl.when(i == 0)
    def _():
        aacc_ref[...] = jnp.zeros((TR, FS), _F32)
        eacc_ref[...] = jnp.zeros((TE, FS), _F32)

    aidx_row = aidx_ref[0]
    eidx_row = eidx_ref[0]

    lane40 = lax.broadcasted_iota(jnp.int32, (B, FS), 1)
    x40a = jnp.where(lane40 < N_AE, x_ref[:, 0:FS], 1.0)
    x40e = jnp.where(lane40 >= 8, x_ref[:, 24:64], 1.0)
    xa_hi, xa_lo = _split_hi_lo(x40a)
    xe_hi, xe_lo = _split_hi_lo(x40e)

    sub_e = lax.broadcasted_iota(jnp.int32, (TE, B), 0)
    ohe = (sub_e == eidx_row).astype(_BF)
    eacc_ref[...] += (jnp.dot(ohe, xe_hi, preferred_element_type=_F32)
                      + jnp.dot(ohe, xe_lo, preferred_element_type=_F32))

    lo = lo_ref[i]
    hi = hi_ref[i]
    base = (lo // 8) * 8
    nwin = (hi - base) // W + 1
    sub_a = lax.broadcasted_iota(jnp.int32, (W, B), 0)

    def wloop(k, _):
        ws = base + k * W
        oh = ((sub_a + ws) == aidx_row).astype(_BF)
        contrib = (jnp.dot(oh, xa_hi, preferred_element_type=_F32)
                   + jnp.dot(oh, xa_lo, preferred_element_type=_F32))
        aacc_ref[pl.ds(ws, W), :] += contrib
        return 0

    lax.fori_loop(0, nwin, wloop, 0)


def _k1b_body(lo_ref, hi_ref, aacc_ref, eacc_ref, dist_ref, aidx_ref, eidx_ref,
              w1_ref, h_ref, stats_ref):
    i = pl.program_id(0)
    aidx_row = aidx_ref[0]
    eidx_row = eidx_ref[0]

    ecnt = jnp.maximum(eacc_ref[:, 0:1], 1.0)
    pe_tab = jnp.maximum(eacc_ref[:, 8:FS] / ecnt, 0.0)
    pt_hi, pt_lo = _split_hi_lo(pe_tab)
    sub_e = lax.broadcasted_iota(jnp.int32, (TE, B), 0)
    ohe = (sub_e == eidx_row).astype(_BF)
    pe_t = (lax.dot_general(pt_hi, ohe, _C00, preferred_element_type=_F32)
            + lax.dot_general(pt_lo, ohe, _C00, preferred_element_type=_F32))

    lo = lo_ref[i]
    hi = hi_ref[i]
    base = (lo // 8) * 8
    nwin = (hi - base) // W + 1
    sub_a = lax.broadcasted_iota(jnp.int32, (W, B), 0)

    def wloop(k, pa_t):
        ws = base + k * W
        win = aacc_ref[pl.ds(ws, W), :]
        cnt = jnp.maximum(win[:, N_AE:N_AE + 1], 1.0)
        ptab = jnp.maximum(win[:, :N_AE] / cnt, 0.0)
        at_hi, at_lo = _split_hi_lo(ptab)
        oh = ((sub_a + ws) == aidx_row).astype(_BF)
        return (pa_t
                + lax.dot_general(at_hi, oh, _C00, preferred_element_type=_F32)
                + lax.dot_general(at_lo, oh, _C00, preferred_element_type=_F32))

    pa_t = lax.fori_loop(0, nwin, wloop, jnp.zeros((N_AE, B), _F32))

    pa = pa_t.T
    pe = pe_t.T
    hb = (jnp.dot(dist_ref[...], w1_ref[0:N_DE, :], preferred_element_type=_F32)
          + jnp.dot(pa, w1_ref[N_DE:N_DE + N_AE, :], preferred_element_type=_F32)
          + jnp.dot(pe, w1_ref[N_DE + N_AE:, :], preferred_element_type=_F32))
    h_ref[...] = hb

    @pl.when(i == 0)
    def _():
        stats_ref[...] = jnp.zeros((8, 128), _F32)

    both = jnp.concatenate([hb, hb * hb], axis=1)
    ones8 = jnp.ones((8, B), _F32)
    stats_ref[...] += jnp.dot(ones8, both, preferred_element_type=_F32)


def _k2_body(h_ref, x_ref, stats_ref, gamma_ref, beta_ref, out_ref):
    inv_n = 1.0 / N_ROWS
    mu = stats_ref[0:1, 0:64] * inv_n
    ex2 = stats_ref[0:1, 64:128] * inv_n
    var = ex2 - mu * mu
    inv = lax.rsqrt(var + 1e-5)
    scale = gamma_ref[...] * inv
    shift = beta_ref[...] - mu * scale
    out_ref[...] = jnp.maximum(h_ref[...] * scale + shift + x_ref[...], 0.0)


@jax.jit
def kernel(x, dist_feat, atom_idx, ele_idx, W1, b1, gamma, beta):
    del b1
    aidx = atom_idx.astype(jnp.int32)
    eidx = ele_idx.astype(jnp.int32)
    lo = aidx[::B]
    hi = aidx[B - 1::B]
    aidx3 = aidx.reshape(NB, 1, B)
    eidx3 = eidx.reshape(NB, 1, B)

    grid_a = pltpu.PrefetchScalarGridSpec(
        num_scalar_prefetch=2,
        grid=(NB,),
        in_specs=[
            pl.BlockSpec((B, 64), lambda i, lo, hi: (i, 0)),
            pl.BlockSpec((1, 1, B), lambda i, lo, hi: (i, 0, 0)),
            pl.BlockSpec((1, 1, B), lambda i, lo, hi: (i, 0, 0)),
        ],
        out_specs=[
            pl.BlockSpec((TR, FS), lambda i, lo, hi: (0, 0)),
            pl.BlockSpec((TE, FS), lambda i, lo, hi: (0, 0)),
        ],
    )
    aacc, eacc = pl.pallas_call(
        _k1a_body,
        grid_spec=grid_a,
        out_shape=[
            jax.ShapeDtypeStruct((TR, FS), _F32),
            jax.ShapeDtypeStruct((TE, FS), _F32),
        ],
        compiler_params=pltpu.CompilerParams(
            dimension_semantics=("arbitrary",),
        ),
    )(lo, hi, x, aidx3, eidx3)

    grid_b = pltpu.PrefetchScalarGridSpec(
        num_scalar_prefetch=2,
        grid=(NB,),
        in_specs=[
            pl.BlockSpec((TR, FS), lambda i, lo, hi: (0, 0)),
            pl.BlockSpec((TE, FS), lambda i, lo, hi: (0, 0)),
            pl.BlockSpec((B, N_DE), lambda i, lo, hi: (i, 0)),
            pl.BlockSpec((1, 1, B), lambda i, lo, hi: (i, 0, 0)),
            pl.BlockSpec((1, 1, B), lambda i, lo, hi: (i, 0, 0)),
            pl.BlockSpec((80, 64), lambda i, lo, hi: (0, 0)),
        ],
        out_specs=[
            pl.BlockSpec((B, 64), lambda i, lo, hi: (i, 0)),
            pl.BlockSpec((8, 128), lambda i, lo, hi: (0, 0)),
        ],
    )
    h, stats = pl.pallas_call(
        _k1b_body,
        grid_spec=grid_b,
        out_shape=[
            jax.ShapeDtypeStruct((N_ROWS, 64), _F32),
            jax.ShapeDtypeStruct((8, 128), _F32),
        ],
        compiler_params=pltpu.CompilerParams(
            dimension_semantics=("arbitrary",),
        ),
    )(lo, hi, aacc, eacc, dist_feat, aidx3, eidx3, W1)

    out = pl.pallas_call(
        _k2_body,
        grid=(NB,),
        in_specs=[
            pl.BlockSpec((B, 64), lambda i: (i, 0)),
            pl.BlockSpec((B, 64), lambda i: (i, 0)),
            pl.BlockSpec((8, 128), lambda i: (0, 0)),
            pl.BlockSpec((1, 64), lambda i: (0, 0)),
            pl.BlockSpec((1, 64), lambda i: (0, 0)),
        ],
        out_specs=pl.BlockSpec((B, 64), lambda i: (i, 0)),
        out_shape=jax.ShapeDtypeStruct((N_ROWS, 64), jnp.float32),
        compiler_params=pltpu.CompilerParams(
            dimension_semantics=("arbitrary",),
        ),
    )(h, x, stats, gamma.reshape(1, 64), beta.reshape(1, 64))
    return out
